# baseline (speedup 1.0000x reference)
import functools

import jax
import jax.numpy as jnp
from jax import lax
from jax.experimental import pallas as pl
from jax.experimental.pallas import tpu as pltpu
from jax.experimental.pallas import tpu_sc as plsc

N_NODES = 10000
N_EDGES = 320000
D = 128

NC = 2
NS = 16
HALF = N_NODES // NC
ACC_ROWS = 5120
DUMMY_ROW = HALF
EPT = N_EDGES // NS
SCAN = EPT // 5
CAP = 20480
C = 64
NB = 3
ZR = 8


def _sc_segment_partials(x, src_e, dst_e):
  mesh = plsc.VectorSubcoreMesh(core_axis_name="c", subcore_axis_name="s")

  @functools.partial(
      pl.kernel,
      out_type=jax.ShapeDtypeStruct((NC, ACC_ROWS, D), jnp.float32),
      mesh=mesh,
      compiler_params=pltpu.CompilerParams(use_tc_tiling_on_sc=False,
                                          needs_layout_passes=False),
      scratch_types=[
          pltpu.VMEM((SCAN,), jnp.int32),
          pltpu.VMEM((SCAN,), jnp.int32),
          pltpu.VMEM((CAP,), jnp.int32),
          pltpu.VMEM((CAP,), jnp.int32),
          pltpu.VMEM((NB, C, D), jnp.float32),
          pltpu.VMEM((NB, C), jnp.int32),
          pltpu.VMEM((ZR, D), jnp.float32),
          pltpu.VMEM_SHARED((ACC_ROWS, D), jnp.float32),
      ] + [pltpu.SemaphoreType.DMA] * (2 * NB),
  )
  def k(x_hbm, src_hbm, dst_hbm, out_hbm, ev_s, ev_d, bsrc, bdst,
        rows_v, dstc, zb, acc, *sems):
    gsem = sems[:NB]
    ssem = sems[NB:]
    cid = lax.axis_index("c")
    sid = lax.axis_index("s")
    base = cid * HALF

    z16 = jnp.zeros((16,), jnp.float32)
    for r in range(ZR):
      for cb in range(D // 16):
        zb[r, pl.ds(cb * 16, 16)] = z16
    rows_per_tile = ACC_ROWS // NS
    for kk in range(rows_per_tile // ZR):
      pltpu.sync_copy(zb, acc.at[pl.ds(sid * rows_per_tile + kk * ZR, ZR)])

    def scan_pass(p, cnt0):
      pltpu.sync_copy(src_hbm.at[sid, pl.ds(p * SCAN, SCAN)], ev_s)
      pltpu.sync_copy(dst_hbm.at[sid, pl.ds(p * SCAN, SCAN)], ev_d)

      base16 = jnp.broadcast_to(base, (16,)).astype(jnp.int32)

      def body(i, cnt):
        s16 = ev_s[pl.ds(i * 16, 16)]
        d16 = ev_d[pl.ds(i * 16, 16)]
        dloc = d16 - base16
        m = (dloc >= 0) & (dloc < HALF)
        mi = m.astype(jnp.int32)
        incl = plsc.cumsum(mi)
        pos = jnp.broadcast_to(cnt - 1, (16,)).astype(jnp.int32) + incl
        plsc.store_scatter(bsrc, [pos], s16, mask=m)
        plsc.store_scatter(bdst, [pos], dloc, mask=m)
        return cnt + jnp.sum(mi)

      return lax.fori_loop(0, SCAN // 16, body, cnt0)

    cnt = jnp.int32(0)
    for p_ in range(EPT // SCAN):
      cnt = scan_pass(p_, cnt)

    zero16 = jnp.zeros((16,), jnp.int32)
    dummy16 = jnp.full((16,), DUMMY_ROW, jnp.int32)
    for kk in range(C // 16):
      bsrc[pl.ds(cnt + kk * 16, 16)] = zero16
      bdst[pl.ds(cnt + kk * 16, 16)] = dummy16
    nch = lax.div(cnt + (C - 1), C)

    plsc.subcore_barrier()

    def issue_gather(j, b):
      pltpu.async_copy(x_hbm.at[bsrc.at[pl.ds(j * C, C)]], rows_v.at[b],
                       gsem[b])

    def wait_gather(j, b):
      pltpu.make_async_copy(x_hbm.at[bsrc.at[pl.ds(j * C, C)]],
                            rows_v.at[b], gsem[b]).wait()

    def issue_scatter(j, b):
      pltpu.async_copy(rows_v.at[b], acc.at[dstc.at[b]], ssem[b], add=True)

    def wait_scatter(j, b):
      pltpu.make_async_copy(rows_v.at[b], acc.at[dstc.at[b]],
                            ssem[b]).wait()

    for b in range(NB):
      @pl.when(b < nch)
      def _():
        issue_gather(b, b)

    def round_body(i, _):
      for b in range(NB):
        j = i * NB + b

        @pl.when(j < nch)
        def _():
          wait_gather(j, b)
          for kk in range(C // 16):
            dstc[b, pl.ds(kk * 16, 16)] = bdst[pl.ds(j * C + kk * 16, 16)]
          issue_scatter(j, b)
          wait_scatter(j, b)

          @pl.when(j + NB < nch)
          def _():
            issue_gather(j + NB, b)

      return 0

    lax.fori_loop(0, lax.div(nch + (NB - 1), NB), round_body, 0)
    plsc.subcore_barrier()

    pltpu.sync_copy(acc.at[pl.ds(sid * rows_per_tile, rows_per_tile)],
                    out_hbm.at[cid, pl.ds(sid * rows_per_tile,
                                          rows_per_tile)])

  return k(x, src_e, dst_e)


def _tc_combine(alpha2d, x, partials, W, b2d):
  BN = 1000
  blocks_per_half = HALF // BN

  def body(al_ref, x_ref, p_ref, w_ref, b_ref, o_ref):
    scale = 1.0 + al_ref[0, 0]
    h = x_ref[...] * scale + p_ref[0]
    o_ref[...] = lax.dot_general(
        h, w_ref[...], (((1,), (1,)), ((), ())),
        preferred_element_type=jnp.float32) + b_ref[...]

  return pl.pallas_call(
      body,
      out_shape=jax.ShapeDtypeStruct((N_NODES, D), jnp.float32),
      grid=(N_NODES // BN,),
      in_specs=[
          pl.BlockSpec((1, 1), lambda i: (0, 0)),
          pl.BlockSpec((BN, D), lambda i: (i, 0)),
          pl.BlockSpec((1, BN, D),
                       lambda i: (i // blocks_per_half,
                                  i % blocks_per_half, 0)),
          pl.BlockSpec((D, D), lambda i: (0, 0)),
          pl.BlockSpec((1, D), lambda i: (0, 0)),
      ],
      out_specs=pl.BlockSpec((BN, D), lambda i: (i, 0)),
  )(alpha2d, x, partials, W, b2d)


def kernel(nfeats, edge_index, W, b, alpha):
  ei = edge_index.astype(jnp.int32)
  src_e = ei[0].reshape(NS, EPT)
  dst_e = ei[1].reshape(NS, EPT)

  partials = _sc_segment_partials(nfeats, src_e, dst_e)

  alpha2d = alpha.reshape(1, 1)
  b2d = b.reshape(1, D)
  return _tc_combine(alpha2d, nfeats, partials, W, b2d)


# speedup vs baseline: 1.0797x; 1.0797x over previous
import functools

import jax
import jax.numpy as jnp
from jax import lax
from jax.experimental import pallas as pl
from jax.experimental.pallas import tpu as pltpu
from jax.experimental.pallas import tpu_sc as plsc

N_NODES = 10000
N_EDGES = 320000
D = 128

NC = 2
NS = 16
HALF = N_NODES // NC
ACC_ROWS = 5120
DUMMY_ROW = HALF
EPT = N_EDGES // NS
SCAN = EPT // 5
CAP = 20480
C = 64
NB = 4
ZR = 8


def _sc_segment_partials(x, edges):
  mesh = plsc.VectorSubcoreMesh(core_axis_name="c", subcore_axis_name="s")

  @functools.partial(
      pl.kernel,
      out_type=jax.ShapeDtypeStruct((NC, ACC_ROWS, D), jnp.float32),
      mesh=mesh,
      compiler_params=pltpu.CompilerParams(use_tc_tiling_on_sc=False,
                                          needs_layout_passes=False),
      scratch_types=[
          pltpu.VMEM((SCAN,), jnp.int32),
          pltpu.VMEM((SCAN,), jnp.int32),
          pltpu.VMEM((CAP,), jnp.int32),
          pltpu.VMEM((CAP,), jnp.int32),
          pltpu.VMEM((NB, C, D), jnp.float32),
          pltpu.VMEM((NB, C), jnp.int32),
          pltpu.VMEM((ZR, D), jnp.float32),
          pltpu.VMEM_SHARED((ACC_ROWS, D), jnp.float32),
      ] + [pltpu.SemaphoreType.DMA] * (2 * NB),
  )
  def k(x_hbm, edges_hbm, out_hbm, ev_s, ev_d, bsrc, bdst,
        rows_v, dstc, zb, acc, *sems):
    gsem = sems[:NB]
    ssem = sems[NB:]
    cid = lax.axis_index("c")
    sid = lax.axis_index("s")
    base = cid * HALF

    z16 = jnp.zeros((16,), jnp.float32)
    for r in range(ZR):
      for cb in range(D // 16):
        zb[r, pl.ds(cb * 16, 16)] = z16
    rows_per_tile = ACC_ROWS // NS
    for kk in range(rows_per_tile // ZR):
      pltpu.sync_copy(zb, acc.at[pl.ds(sid * rows_per_tile + kk * ZR, ZR)])

    def scan_pass(p, cnt0):
      off = sid * EPT + p * SCAN
      pltpu.sync_copy(edges_hbm.at[0, pl.ds(off, SCAN)], ev_s)
      pltpu.sync_copy(edges_hbm.at[1, pl.ds(off, SCAN)], ev_d)

      base16 = jnp.broadcast_to(base, (16,)).astype(jnp.int32)

      def body(i, cnt):
        s16 = ev_s[pl.ds(i * 16, 16)]
        d16 = ev_d[pl.ds(i * 16, 16)]
        dloc = d16 - base16
        m = (dloc >= 0) & (dloc < HALF)
        mi = m.astype(jnp.int32)
        incl = plsc.cumsum(mi)
        pos = jnp.broadcast_to(cnt - 1, (16,)).astype(jnp.int32) + incl
        plsc.store_scatter(bsrc, [pos], s16, mask=m)
        plsc.store_scatter(bdst, [pos], dloc, mask=m)
        return cnt + jnp.sum(mi)

      return lax.fori_loop(0, SCAN // 16, body, cnt0)

    cnt = jnp.int32(0)
    for p_ in range(EPT // SCAN):
      cnt = scan_pass(p_, cnt)

    zero16 = jnp.zeros((16,), jnp.int32)
    dummy16 = jnp.full((16,), DUMMY_ROW, jnp.int32)
    for kk in range(C // 16):
      bsrc[pl.ds(cnt + kk * 16, 16)] = zero16
      bdst[pl.ds(cnt + kk * 16, 16)] = dummy16
    nch = lax.div(cnt + (C - 1), C)

    plsc.subcore_barrier()

    def issue_gather(j, b):
      pltpu.async_copy(x_hbm.at[bsrc.at[pl.ds(j * C, C)]], rows_v.at[b],
                       gsem[b])

    def wait_gather(j, b):
      pltpu.make_async_copy(x_hbm.at[bsrc.at[pl.ds(j * C, C)]],
                            rows_v.at[b], gsem[b]).wait()

    def issue_scatter(j, b):
      pltpu.async_copy(rows_v.at[b], acc.at[dstc.at[b]], ssem[b], add=True)

    def wait_scatter(j, b):
      pltpu.make_async_copy(rows_v.at[b], acc.at[dstc.at[b]],
                            ssem[b]).wait()

    for b in range(NB):
      @pl.when(b < nch)
      def _():
        issue_gather(b, b)

    def round_body(i, _):
      for b in range(NB):
        j = i * NB + b

        @pl.when(j < nch)
        def _():
          wait_gather(j, b)
          for kk in range(C // 16):
            dstc[b, pl.ds(kk * 16, 16)] = bdst[pl.ds(j * C + kk * 16, 16)]
          issue_scatter(j, b)
          wait_scatter(j, b)

          @pl.when(j + NB < nch)
          def _():
            issue_gather(j + NB, b)

      return 0

    lax.fori_loop(0, lax.div(nch + (NB - 1), NB), round_body, 0)
    plsc.subcore_barrier()

    pltpu.sync_copy(acc.at[pl.ds(sid * rows_per_tile, rows_per_tile)],
                    out_hbm.at[cid, pl.ds(sid * rows_per_tile,
                                          rows_per_tile)])

  return k(x, edges)


def _tc_combine(alpha2d, x, partials, W, b2d):
  BN = 1000
  blocks_per_half = HALF // BN

  def body(al_ref, x_ref, p_ref, w_ref, b_ref, o_ref):
    scale = 1.0 + al_ref[0, 0]
    h = x_ref[...] * scale + p_ref[0]
    o_ref[...] = lax.dot_general(
        h, w_ref[...], (((1,), (1,)), ((), ())),
        preferred_element_type=jnp.float32) + b_ref[...]

  return pl.pallas_call(
      body,
      out_shape=jax.ShapeDtypeStruct((N_NODES, D), jnp.float32),
      grid=(N_NODES // BN,),
      in_specs=[
          pl.BlockSpec((1, 1), lambda i: (0, 0)),
          pl.BlockSpec((BN, D), lambda i: (i, 0)),
          pl.BlockSpec((1, BN, D),
                       lambda i: (i // blocks_per_half,
                                  i % blocks_per_half, 0)),
          pl.BlockSpec((D, D), lambda i: (0, 0)),
          pl.BlockSpec((1, D), lambda i: (0, 0)),
      ],
      out_specs=pl.BlockSpec((BN, D), lambda i: (i, 0)),
  )(alpha2d, x, partials, W, b2d)


def kernel(nfeats, edge_index, W, b, alpha):
  ei = edge_index.astype(jnp.int32)

  partials = _sc_segment_partials(nfeats, ei)

  alpha2d = alpha.reshape(1, 1)
  b2d = b.reshape(1, D)
  return _tc_combine(alpha2d, nfeats, partials, W, b2d)
